# Initial kernel scaffold; baseline (speedup 1.0000x reference)
import jax
import jax.numpy as jnp
from jax.experimental import pallas as pl
from jax.experimental.pallas import tpu as pltpu
from jax.experimental.pallas import tpu_sc as plsc

VOCAB = 100000
D = 128
N = 2048
TILE_V = 1024
NT = (VOCAB + TILE_V - 1) // TILE_V
GATHER_WINDOW = 128


def _emb_gather(emb, tokens):
    mesh = plsc.VectorSubcoreMesh(core_axis_name="core",
                                  subcore_axis_name="subcore")

    @pl.kernel(out_type=jax.ShapeDtypeStruct((N, D), emb.dtype), mesh=mesh)
    def gather_kernel(emb_hbm, idx_hbm, out_hbm):
        def body(idx_vmem, out_vmem):
            pltpu.sync_copy(emb_hbm.at[idx_vmem.at[0]], out_vmem)

        pltpu.emit_pipeline(
            body,
            grid=(N // GATHER_WINDOW,),
            in_specs=[pl.BlockSpec((1, GATHER_WINDOW),
                                   index_map=lambda i: (0, i))],
            out_specs=[pl.BlockSpec((GATHER_WINDOW, D),
                                    index_map=lambda i: (i, 0))],
            core_axis_name="subcore",
            dimension_semantics=(pltpu.PARALLEL,),
        )(idx_hbm, out_hbm)

    return gather_kernel(emb, tokens)


def _loss_body(h0_ref, w_ref, wout_ref, lab_ref, out_ref,
               h_scr, m_scr, s_scr, p_scr):
    i = pl.program_id(0)

    @pl.when(i == 0)
    def _():
        h_scr[...] = jnp.tanh(
            jnp.dot(h0_ref[...], w_ref[...],
                    preferred_element_type=jnp.float32))
        m_scr[...] = jnp.full((N, 1), -1e30, jnp.float32)
        s_scr[...] = jnp.zeros((N, 1), jnp.float32)
        p_scr[...] = jnp.zeros((N, 1), jnp.float32)

    lg = jnp.dot(h_scr[...], wout_ref[...],
                 preferred_element_type=jnp.float32)
    ids = i * TILE_V + jax.lax.broadcasted_iota(jnp.int32, (N, TILE_V), 1)
    lg = jnp.where(ids < VOCAB, lg, -1e30)

    m_old = m_scr[...]
    m_new = jnp.maximum(m_old, jnp.max(lg, axis=1, keepdims=True))
    s_scr[...] = (s_scr[...] * jnp.exp(m_old - m_new)
                  + jnp.sum(jnp.exp(lg - m_new), axis=1, keepdims=True))
    m_scr[...] = m_new
    p_scr[...] += jnp.sum(jnp.where(lab_ref[...] == ids, lg, 0.0),
                          axis=1, keepdims=True)

    @pl.when(i == NT - 1)
    def _():
        nll = m_scr[...] + jnp.log(s_scr[...]) - p_scr[...]
        rows = jax.lax.broadcasted_iota(jnp.int32, (N, 1), 0)
        nll = jnp.where(rows < N - 1, nll, 0.0)
        out_ref[0, 0] = jnp.sum(nll) / (N - 1)


def kernel(x, emb, W, Wout):
    h0 = _emb_gather(emb, x)
    labels = jnp.concatenate(
        [x[0, 1:], jnp.zeros((1,), jnp.int32)]).reshape(N, 1)

    out = pl.pallas_call(
        _loss_body,
        grid=(NT,),
        in_specs=[
            pl.BlockSpec((N, D), lambda i: (0, 0)),
            pl.BlockSpec((D, D), lambda i: (0, 0)),
            pl.BlockSpec((D, TILE_V), lambda i: (0, i)),
            pl.BlockSpec((N, 1), lambda i: (0, 0)),
        ],
        out_specs=pl.BlockSpec((1, 1), lambda i: (0, 0)),
        out_shape=jax.ShapeDtypeStruct((1, 1), jnp.float32),
        scratch_shapes=[
            pltpu.VMEM((N, D), jnp.float32),
            pltpu.VMEM((N, 1), jnp.float32),
            pltpu.VMEM((N, 1), jnp.float32),
            pltpu.VMEM((N, 1), jnp.float32),
        ],
    )(h0, W, Wout, labels)
    return out[0, 0]


# speedup vs baseline: 1.8700x; 1.8700x over previous
import jax
import jax.numpy as jnp
from jax.experimental import pallas as pl
from jax.experimental.pallas import tpu as pltpu
from jax.experimental.pallas import tpu_sc as plsc

VOCAB = 100000
D = 128
N = 2048
TILE_V = 1024
NT = (VOCAB + TILE_V - 1) // TILE_V
GATHER_WINDOW = 128


def _emb_gather(emb, tokens):
    mesh = plsc.VectorSubcoreMesh(core_axis_name="core",
                                  subcore_axis_name="subcore")

    @pl.kernel(out_type=jax.ShapeDtypeStruct((N, D), emb.dtype), mesh=mesh)
    def gather_kernel(emb_hbm, idx_hbm, out_hbm):
        def body(idx_vmem, out_vmem):
            pltpu.sync_copy(emb_hbm.at[idx_vmem.at[0]], out_vmem)

        pltpu.emit_pipeline(
            body,
            grid=(N // GATHER_WINDOW,),
            in_specs=[pl.BlockSpec((1, GATHER_WINDOW),
                                   index_map=lambda i: (0, i))],
            out_specs=[pl.BlockSpec((GATHER_WINDOW, D),
                                    index_map=lambda i: (i, 0))],
            core_axis_name="subcore",
            dimension_semantics=(pltpu.PARALLEL,),
        )(idx_hbm, out_hbm)

    return gather_kernel(emb, tokens)


def _loss_body(h0_ref, w_ref, wout_ref, lab_ref, out_ref,
               h_scr, m_scr, s_scr, p_scr):
    i = pl.program_id(0)

    @pl.when(i == 0)
    def _():
        h_scr[...] = jnp.tanh(
            jnp.dot(h0_ref[...], w_ref[...],
                    preferred_element_type=jnp.float32))
        m_scr[...] = jnp.full((N, 1), -1e30, jnp.float32)
        s_scr[...] = jnp.zeros((N, 1), jnp.float32)
        p_scr[...] = jnp.zeros((N, 1), jnp.float32)

    lg = jnp.dot(h_scr[...], wout_ref[...],
                 preferred_element_type=jnp.float32)
    ids = i * TILE_V + jax.lax.broadcasted_iota(jnp.int32, (N, TILE_V), 1)
    lg = jnp.where(ids < VOCAB, lg, -1e30)

    m_old = m_scr[...]
    m_new = jnp.maximum(m_old, jnp.max(lg, axis=1, keepdims=True))
    s_scr[...] = (s_scr[...] * jnp.exp(m_old - m_new)
                  + jnp.sum(jnp.exp(lg - m_new), axis=1, keepdims=True))
    m_scr[...] = m_new
    p_scr[...] += jnp.sum(jnp.where(lab_ref[...] == ids, lg, 0.0),
                          axis=1, keepdims=True)

    @pl.when(i == NT - 1)
    def _():
        nll = m_scr[...] + jnp.log(s_scr[...]) - p_scr[...]
        rows = jax.lax.broadcasted_iota(jnp.int32, (N, 1), 0)
        nll = jnp.where(rows < N - 1, nll, 0.0)
        out_ref[...] = (jnp.sum(nll) / (N - 1)).reshape(1, 1)


def kernel(x, emb, W, Wout):
    h0 = _emb_gather(emb, x)
    labels = jnp.concatenate(
        [x[0, 1:], jnp.zeros((1,), jnp.int32)]).reshape(N, 1)

    out = pl.pallas_call(
        _loss_body,
        grid=(NT,),
        in_specs=[
            pl.BlockSpec((N, D), lambda i: (0, 0)),
            pl.BlockSpec((D, D), lambda i: (0, 0)),
            pl.BlockSpec((D, TILE_V), lambda i: (0, i)),
            pl.BlockSpec((N, 1), lambda i: (0, 0)),
        ],
        out_specs=pl.BlockSpec((1, 1), lambda i: (0, 0)),
        out_shape=jax.ShapeDtypeStruct((1, 1), jnp.float32),
        scratch_shapes=[
            pltpu.VMEM((N, D), jnp.float32),
            pltpu.VMEM((N, 1), jnp.float32),
            pltpu.VMEM((N, 1), jnp.float32),
            pltpu.VMEM((N, 1), jnp.float32),
        ],
    )(h0, W, Wout, labels)
    return out[0, 0]


# speedup vs baseline: 1.8719x; 1.0010x over previous
import jax
import jax.numpy as jnp
from jax.experimental import pallas as pl
from jax.experimental.pallas import tpu as pltpu
from jax.experimental.pallas import tpu_sc as plsc

VOCAB = 100000
D = 128
N = 2048
TILE_V = 1024
NT = (VOCAB + TILE_V - 1) // TILE_V
GATHER_WINDOW = 128


def _emb_gather(emb, tokens):
    mesh = plsc.VectorSubcoreMesh(core_axis_name="core",
                                  subcore_axis_name="subcore")

    @pl.kernel(out_type=jax.ShapeDtypeStruct((N, D), emb.dtype), mesh=mesh)
    def gather_kernel(emb_hbm, idx_hbm, out_hbm):
        def body(idx_vmem, out_vmem):
            pltpu.sync_copy(emb_hbm.at[idx_vmem.at[0]], out_vmem)

        pltpu.emit_pipeline(
            body,
            grid=(N // GATHER_WINDOW,),
            in_specs=[pl.BlockSpec((1, GATHER_WINDOW),
                                   index_map=lambda i: (0, i))],
            out_specs=[pl.BlockSpec((GATHER_WINDOW, D),
                                    index_map=lambda i: (i, 0))],
            core_axis_name="subcore",
            dimension_semantics=(pltpu.PARALLEL,),
        )(idx_hbm, out_hbm)

    return gather_kernel(emb, tokens)


def _loss_body(h0_ref, w_ref, wout_ref, lab_ref, out_ref,
               h_scr, m_scr, s_scr, p_scr):
    i = pl.program_id(0)

    @pl.when(i == 0)
    def _():
        h_scr[...] = jnp.tanh(
            jnp.dot(h0_ref[...], w_ref[...],
                    preferred_element_type=jnp.float32)).astype(jnp.bfloat16)
        m_scr[...] = jnp.full((N, 1), -1e30, jnp.float32)
        s_scr[...] = jnp.zeros((N, 1), jnp.float32)
        p_scr[...] = jnp.zeros((N, 1), jnp.float32)

    lg = jnp.dot(h_scr[...], wout_ref[...].astype(jnp.bfloat16),
                 preferred_element_type=jnp.float32)
    ids = i * TILE_V + jax.lax.broadcasted_iota(jnp.int32, (N, TILE_V), 1)
    lg = jnp.where(ids < VOCAB, lg, -1e30)

    m_old = m_scr[...]
    m_new = jnp.maximum(m_old, jnp.max(lg, axis=1, keepdims=True))
    s_scr[...] = (s_scr[...] * jnp.exp(m_old - m_new)
                  + jnp.sum(jnp.exp(lg - m_new), axis=1, keepdims=True))
    m_scr[...] = m_new
    p_scr[...] += jnp.sum(jnp.where(lab_ref[...] == ids, lg, 0.0),
                          axis=1, keepdims=True)

    @pl.when(i == NT - 1)
    def _():
        nll = m_scr[...] + jnp.log(s_scr[...]) - p_scr[...]
        rows = jax.lax.broadcasted_iota(jnp.int32, (N, 1), 0)
        nll = jnp.where(rows < N - 1, nll, 0.0)
        out_ref[...] = (jnp.sum(nll) / (N - 1)).reshape(1, 1)


def kernel(x, emb, W, Wout):
    h0 = _emb_gather(emb, x)
    labels = jnp.concatenate(
        [x[0, 1:], jnp.zeros((1,), jnp.int32)]).reshape(N, 1)

    out = pl.pallas_call(
        _loss_body,
        grid=(NT,),
        in_specs=[
            pl.BlockSpec((N, D), lambda i: (0, 0)),
            pl.BlockSpec((D, D), lambda i: (0, 0)),
            pl.BlockSpec((D, TILE_V), lambda i: (0, i)),
            pl.BlockSpec((N, 1), lambda i: (0, 0)),
        ],
        out_specs=pl.BlockSpec((1, 1), lambda i: (0, 0)),
        out_shape=jax.ShapeDtypeStruct((1, 1), jnp.float32),
        scratch_shapes=[
            pltpu.VMEM((N, D), jnp.bfloat16),
            pltpu.VMEM((N, 1), jnp.float32),
            pltpu.VMEM((N, 1), jnp.float32),
            pltpu.VMEM((N, 1), jnp.float32),
        ],
    )(h0, W, Wout, labels)
    return out[0, 0]


# speedup vs baseline: 3.8744x; 2.0697x over previous
import jax
import jax.numpy as jnp
from jax.experimental import pallas as pl
from jax.experimental.pallas import tpu as pltpu
from jax.experimental.pallas import tpu_sc as plsc

VOCAB = 100000
D = 128
N = 2048
TILE_V = 1024
NT = (VOCAB + TILE_V - 1) // TILE_V
GATHER_WINDOW = 128


def _emb_gather(emb, tokens):
    mesh = plsc.VectorSubcoreMesh(core_axis_name="core",
                                  subcore_axis_name="subcore")

    @pl.kernel(out_type=jax.ShapeDtypeStruct((N, D), emb.dtype), mesh=mesh)
    def gather_kernel(emb_hbm, idx_hbm, out_hbm):
        def body(idx_vmem, out_vmem):
            pltpu.sync_copy(emb_hbm.at[idx_vmem.at[0]], out_vmem)

        pltpu.emit_pipeline(
            body,
            grid=(N // GATHER_WINDOW,),
            in_specs=[pl.BlockSpec((1, GATHER_WINDOW),
                                   index_map=lambda i: (0, i))],
            out_specs=[pl.BlockSpec((GATHER_WINDOW, D),
                                    index_map=lambda i: (i, 0))],
            core_axis_name="subcore",
            dimension_semantics=(pltpu.PARALLEL,),
        )(idx_hbm, out_hbm)

    return gather_kernel(emb, tokens)


def _loss_body(h0_ref, w_ref, wout_ref, lab_ref, out_ref,
               h_scr, s_scr, p_scr):
    i = pl.program_id(0)

    @pl.when(i == 0)
    def _():
        h_scr[...] = jnp.tanh(
            jnp.dot(h0_ref[...], w_ref[...],
                    preferred_element_type=jnp.float32)).astype(jnp.bfloat16)
        s_scr[...] = jnp.zeros((N, D), jnp.float32)
        p_scr[...] = jnp.zeros((N, D), jnp.float32)

    lg = jnp.dot(h_scr[...], wout_ref[...].astype(jnp.bfloat16),
                 preferred_element_type=jnp.float32)

    ids = i * TILE_V + jax.lax.broadcasted_iota(jnp.int32, (N, TILE_V), 1)
    lg = jnp.where(ids < VOCAB, lg, -1e30)
    e = jnp.exp(lg)
    pk = jnp.where(lab_ref[...] == ids, lg, 0.0)

    s_part = jnp.zeros((N, D), jnp.float32)
    p_part = jnp.zeros((N, D), jnp.float32)
    for g in range(TILE_V // D):
        s_part = s_part + e[:, g * D:(g + 1) * D]
        p_part = p_part + pk[:, g * D:(g + 1) * D]
    s_scr[...] += s_part
    p_scr[...] += p_part

    @pl.when(i == NT - 1)
    def _():
        s_row = jnp.sum(s_scr[...], axis=1, keepdims=True)
        p_row = jnp.sum(p_scr[...], axis=1, keepdims=True)
        nll = jnp.log(s_row) - p_row
        rows = jax.lax.broadcasted_iota(jnp.int32, (N, 1), 0)
        nll = jnp.where(rows < N - 1, nll, 0.0)
        out_ref[...] = (jnp.sum(nll) / (N - 1)).reshape(1, 1)


def kernel(x, emb, W, Wout):
    h0 = _emb_gather(emb, x)
    labels = jnp.concatenate(
        [x[0, 1:], jnp.zeros((1,), jnp.int32)]).reshape(N, 1)

    out = pl.pallas_call(
        _loss_body,
        grid=(NT,),
        in_specs=[
            pl.BlockSpec((N, D), lambda i: (0, 0)),
            pl.BlockSpec((D, D), lambda i: (0, 0)),
            pl.BlockSpec((D, TILE_V), lambda i: (0, i)),
            pl.BlockSpec((N, 1), lambda i: (0, 0)),
        ],
        out_specs=pl.BlockSpec((1, 1), lambda i: (0, 0)),
        out_shape=jax.ShapeDtypeStruct((1, 1), jnp.float32),
        scratch_shapes=[
            pltpu.VMEM((N, D), jnp.bfloat16),
            pltpu.VMEM((N, D), jnp.float32),
            pltpu.VMEM((N, D), jnp.float32),
        ],
    )(h0, W, Wout, labels)
    return out[0, 0]
